# Initial kernel scaffold; baseline (speedup 1.0000x reference)
import jax
import jax.numpy as jnp
from jax.experimental import pallas as pl


def kernel(distances, table):
    raise NotImplementedError("write your pallas kernel here")


# speedup vs baseline: 9.0332x; 9.0332x over previous
import jax
import jax.numpy as jnp
from jax import lax
from jax.experimental import pallas as pl
from jax.experimental.pallas import tpu as pltpu
from jax.experimental.pallas import tpu_sc as plsc

N = 1024
H = 16
V = 32
E = N * N

NC = 2
NS = 16
L = 16
NW = NC * NS
PER_W = E // NW
CHUNK = 2048
NCHUNK = PER_W // CHUNK


def _sc_body(dist_hbm, tab_hbm, out_hbm, idx_v, tab_v, out_v, sem):
    wid = lax.axis_index("s") * NC + lax.axis_index("c")
    base = wid * PER_W
    pltpu.sync_copy(tab_hbm, tab_v)

    def chunk_body(c, carry):
        off = base + c * CHUNK
        pltpu.sync_copy(dist_hbm.at[pl.ds(off, CHUNK)], idx_v)

        def grp_body(g, carry2):
            s = pl.multiple_of(g * L, L)
            d = idx_v[pl.ds(s, L)] * H
            for h in range(H):
                out_v[h, pl.ds(s, L)] = plsc.load_gather(tab_v, [d + h])
            return carry2

        lax.fori_loop(0, CHUNK // L, grp_body, 0)
        copies = [
            pltpu.make_async_copy(
                out_v.at[h], out_hbm.at[h, pl.ds(off, CHUNK)], sem)
            for h in range(H)
        ]
        for cp in copies:
            cp.start()
        for cp in copies:
            cp.wait()
        return carry

    lax.fori_loop(0, NCHUNK, chunk_body, 0)


def kernel(distances, table):
    dist_flat = distances.reshape(E).astype(jnp.int32)
    tab_flat = table.reshape(V * H)
    k = pl.kernel(
        _sc_body,
        out_type=jax.ShapeDtypeStruct((H, E), jnp.float32),
        mesh=plsc.VectorSubcoreMesh(core_axis_name="c", subcore_axis_name="s"),
        compiler_params=pltpu.CompilerParams(needs_layout_passes=False),
        scratch_types=[
            pltpu.VMEM((CHUNK,), jnp.int32),
            pltpu.VMEM((V * H,), jnp.float32),
            pltpu.VMEM((H, CHUNK), jnp.float32),
            pltpu.SemaphoreType.DMA,
        ],
    )
    out = k(dist_flat, tab_flat)
    return out.reshape(H, N, N)


# speedup vs baseline: 15.4755x; 1.7132x over previous
import jax
import jax.numpy as jnp
from jax import lax
from jax.experimental import pallas as pl
from jax.experimental.pallas import tpu as pltpu
from jax.experimental.pallas import tpu_sc as plsc

N = 1024
H = 16
V = 32
E = N * N

NC = 2
NS = 16
L = 16
NW = NC * NS
PER_W = E // NW
CHUNK = 2048
NCHUNK = PER_W // CHUNK


def _sc_body(dist_hbm, tab_hbm, out_hbm, idx_v, tab_v, out_v, sem):
    wid = lax.axis_index("s") * NC + lax.axis_index("c")
    base = wid * PER_W
    pltpu.sync_copy(tab_hbm, tab_v)

    def chunk_body(c, carry):
        off = base + c * CHUNK
        pltpu.sync_copy(dist_hbm.at[pl.ds(off, CHUNK)], idx_v)

        def grp_body(g, carry2):
            s = pl.multiple_of(g * L, L)
            d = idx_v[pl.ds(s, L)] * H
            vals = [plsc.load_gather(tab_v, [d + h]) for h in range(H)]
            for h in range(H):
                out_v[h, pl.ds(s, L)] = vals[h]
            return carry2

        lax.fori_loop(0, CHUNK // L, grp_body, 0)
        copies = [
            pltpu.make_async_copy(
                out_v.at[h], out_hbm.at[h, pl.ds(off, CHUNK)], sem)
            for h in range(H)
        ]
        for cp in copies:
            cp.start()
        for cp in copies:
            cp.wait()
        return carry

    lax.fori_loop(0, NCHUNK, chunk_body, 0)


def kernel(distances, table):
    dist_flat = distances.reshape(E).astype(jnp.int32)
    tab_flat = table.reshape(V * H)
    k = pl.kernel(
        _sc_body,
        out_type=jax.ShapeDtypeStruct((H, E), jnp.float32),
        mesh=plsc.VectorSubcoreMesh(core_axis_name="c", subcore_axis_name="s"),
        compiler_params=pltpu.CompilerParams(needs_layout_passes=False),
        scratch_types=[
            pltpu.VMEM((CHUNK,), jnp.int32),
            pltpu.VMEM((V * H,), jnp.float32),
            pltpu.VMEM((H, CHUNK), jnp.float32),
            pltpu.SemaphoreType.DMA,
        ],
    )
    out = k(dist_flat, tab_flat)
    return out.reshape(H, N, N)
